# baseline (speedup 1.0000x reference)
import functools

import jax
import jax.numpy as jnp
from jax import lax
from jax.experimental import pallas as pl
from jax.experimental.pallas import tpu as pltpu
from jax.experimental.pallas import tpu_sc as plsc

B, T, D = 1, 2048, 768
E, K, FF = 8, 2, 1024
BT = B * T
TILE = 512
N_TILES = 15
N_BUF = N_TILES * TILE
NW = 32
ROWS_W = K * BT // NW


def _router_body(x_ref, gate_ref, slots_ref, wbc_ref, base_ref, aux_ref):
    x = x_ref[...]
    logits = lax.dot_general(x, gate_ref[...], (((1,), (1,)), ((), ())),
                             preferred_element_type=jnp.float32)
    m = jnp.max(logits, axis=1, keepdims=True)
    ex = jnp.exp(logits - m)
    s = ex / jnp.sum(ex, axis=1, keepdims=True)
    eids = lax.broadcasted_iota(jnp.int32, (BT, E), 1)
    v1 = jnp.max(s, axis=1, keepdims=True)
    i1 = jnp.min(jnp.where(s == v1, eids, E), axis=1, keepdims=True)
    s_m = jnp.where(eids == i1, -1.0, s)
    v2 = jnp.max(s_m, axis=1, keepdims=True)
    i2 = jnp.min(jnp.where(s_m == v2, eids, E), axis=1, keepdims=True)
    sel = (eids == i1) | (eids == i2)
    self32 = sel.astype(jnp.float32)

    wbc_ref[...] = jnp.concatenate(
        [jnp.broadcast_to(v1, (BT, 16)), jnp.broadcast_to(v2, (BT, 16)),
         jnp.broadcast_to(v1 + v2, (BT, 16))], axis=1)

    load = jnp.sum(self32, axis=0, keepdims=True)
    importance = jnp.sum(s, axis=0, keepdims=True)
    aux = jnp.sum(load * importance) * (E / (BT * BT))
    aux_ref[...] = jnp.broadcast_to(aux, (1, 1))

    cnt = load
    pc = jnp.floor((cnt + (TILE - 1)) / TILE) * TILE
    tri8 = (lax.broadcasted_iota(jnp.int32, (E, E), 0)
            < lax.broadcasted_iota(jnp.int32, (E, E), 1)).astype(jnp.float32)
    base = lax.dot_general(pc, tri8, (((1,), (0,)), ((), ())),
                           preferred_element_type=jnp.float32)
    base_ref[...] = base.astype(jnp.int32)

    ltri = (lax.broadcasted_iota(jnp.int32, (1, BT), 1)
            < lax.broadcasted_iota(jnp.int32, (BT, 1), 0)).astype(jnp.bfloat16)
    p_excl = lax.dot_general(ltri, self32.astype(jnp.bfloat16),
                             (((1,), (0,)), ((), ())),
                             preferred_element_type=jnp.float32)
    slotf = p_excl + base
    slot1 = jnp.sum(jnp.where(eids == i1, slotf, 0.0), axis=1, keepdims=True)
    slot2 = jnp.sum(jnp.where(eids == i2, slotf, 0.0), axis=1, keepdims=True)
    slots_ref[...] = jnp.concatenate(
        [slot1.astype(jnp.int32).reshape(NW, TOK_W),
         slot2.astype(jnp.int32).reshape(NW, TOK_W)], axis=1)


def _router(xf, gate_w):
    return pl.pallas_call(
        _router_body,
        out_specs=[
            pl.BlockSpec((NW, K * TOK_W), lambda: (0, 0)),
            pl.BlockSpec((BT, 48), lambda: (0, 0)),
            pl.BlockSpec((1, E), lambda: (0, 0)),
            pl.BlockSpec((1, 1), lambda: (0, 0)),
        ],
        out_shape=[
            jax.ShapeDtypeStruct((NW, K * TOK_W), jnp.int32),
            jax.ShapeDtypeStruct((BT, 48), jnp.float32),
            jax.ShapeDtypeStruct((1, E), jnp.int32),
            jax.ShapeDtypeStruct((1, 1), jnp.float32),
        ],
    )(xf, gate_w)


TOK_W = BT // NW
HALF = TOK_W // 2
NCH = D // 16


@functools.cache
def _sc_kernels():
    mesh = plsc.VectorSubcoreMesh(core_axis_name="c", subcore_axis_name="s")

    @functools.partial(
        pl.kernel, mesh=mesh,
        out_type=jax.ShapeDtypeStruct((N_BUF, D), jnp.float32),
        scratch_types=[
            pltpu.VMEM((1, K * TOK_W), jnp.int32),
            pltpu.VMEM((TOK_W, D), jnp.float32),
            pltpu.SemaphoreType.DMA,
        ],
    )
    def _sc_scatter(xf_hbm, slot2d_hbm, xs_hbm, idx_v, rows_v, sem):
        wid = lax.axis_index("s") * 2 + lax.axis_index("c")
        t0 = wid * TOK_W
        ci = pltpu.async_copy(slot2d_hbm.at[pl.ds(wid, 1)], idx_v, sem)
        cr = pltpu.async_copy(xf_hbm.at[pl.ds(t0, TOK_W)], rows_v, sem)
        ci.wait()
        cr.wait()
        c1 = pltpu.async_copy(rows_v, xs_hbm.at[idx_v.at[0, pl.ds(0, TOK_W)]],
                              sem)
        c2 = pltpu.async_copy(rows_v,
                              xs_hbm.at[idx_v.at[0, pl.ds(TOK_W, TOK_W)]],
                              sem)
        c1.wait()
        c2.wait()

    @functools.partial(
        pl.kernel, mesh=mesh,
        out_type=jax.ShapeDtypeStruct((BT, D), jnp.float32),
        scratch_types=[
            pltpu.VMEM((1, K * TOK_W), jnp.int32),
            pltpu.VMEM((HALF, D), jnp.float32),
            pltpu.VMEM((HALF, D), jnp.float32),
            pltpu.VMEM((HALF, D), jnp.float32),
            pltpu.VMEM((TOK_W, 48), jnp.float32),
            pltpu.SemaphoreType.DMA,
        ],
    )
    def _sc_combine(ys_hbm, slot2d_hbm, wbc_hbm, xf_hbm, out_hbm,
                    idx_v, r1_v, r2_v, xr_v, wb_v, sem):
        wid = lax.axis_index("s") * 2 + lax.axis_index("c")
        t0 = wid * TOK_W
        pltpu.sync_copy(slot2d_hbm.at[pl.ds(wid, 1)], idx_v)
        pltpu.sync_copy(wbc_hbm.at[pl.ds(t0, TOK_W)], wb_v)
        for half in range(2):
            hb = half * HALF
            c1 = pltpu.async_copy(
                ys_hbm.at[idx_v.at[0, pl.ds(hb, HALF)]], r1_v, sem)
            c2 = pltpu.async_copy(
                ys_hbm.at[idx_v.at[0, pl.ds(TOK_W + hb, HALF)]], r2_v, sem)
            pltpu.sync_copy(xf_hbm.at[pl.ds(t0 + hb, HALF)], xr_v)
            c1.wait()
            c2.wait()

            def row_body(r, carry):
                rw = hb + r
                w1b = wb_v[rw, pl.ds(0, 16)]
                w2b = wb_v[rw, pl.ds(16, 16)]
                wtb = wb_v[rw, pl.ds(32, 16)]
                for c in range(NCH):
                    sl = pl.ds(c * 16, 16)
                    acc = (w1b * r1_v[r, sl] + w2b * r2_v[r, sl]
                           + wtb * xr_v[r, sl])
                    r1_v[r, sl] = acc
                return carry

            lax.fori_loop(0, HALF, row_body, 0)
            pltpu.sync_copy(r1_v, out_hbm.at[pl.ds(t0 + hb, HALF)])

    return _sc_scatter, _sc_combine


def _ffn_body(te_ref, xs_ref, lng_ref, lnb_ref, w1_ref, b1_ref, w2_ref,
              b2_ref, ys_ref):
    j = pl.program_id(0)
    te = te_ref[j]
    lng = lng_ref[pl.ds(te, 1), :]
    lnb = lnb_ref[pl.ds(te, 1), :]
    b1 = b1_ref[pl.ds(te, 1), :]
    b2 = b2_ref[pl.ds(te, 1), :]
    xr = xs_ref[...]
    mu = jnp.mean(xr, axis=1, keepdims=True)
    var = jnp.mean((xr - mu) ** 2, axis=1, keepdims=True)
    xn = ((xr - mu) * lax.rsqrt(var + 1e-6)) * lng + lnb
    h = lax.dot_general(xn.astype(jnp.bfloat16),
                        w1_ref[0].astype(jnp.bfloat16),
                        (((1,), (1,)), ((), ())),
                        preferred_element_type=jnp.float32)
    h = jnp.maximum(h + b1, 0.0)
    y = lax.dot_general(h.astype(jnp.bfloat16),
                        w2_ref[0].astype(jnp.bfloat16),
                        (((1,), (1,)), ((), ())),
                        preferred_element_type=jnp.float32)
    ys_ref[...] = y + b2


def _ffn(te, xs, ln_g, ln_b, W1, b1, W2, b2):
    grid_spec = pltpu.PrefetchScalarGridSpec(
        num_scalar_prefetch=1,
        grid=(N_TILES,),
        in_specs=[
            pl.BlockSpec((TILE, D), lambda j, te: (j, 0)),
            pl.BlockSpec((E, D), lambda j, te: (0, 0)),
            pl.BlockSpec((E, D), lambda j, te: (0, 0)),
            pl.BlockSpec((1, FF, D), lambda j, te: (te[j], 0, 0)),
            pl.BlockSpec((E, FF), lambda j, te: (0, 0)),
            pl.BlockSpec((1, D, FF), lambda j, te: (te[j], 0, 0)),
            pl.BlockSpec((E, D), lambda j, te: (0, 0)),
        ],
        out_specs=pl.BlockSpec((TILE, D), lambda j, te: (j, 0)),
    )
    return pl.pallas_call(
        _ffn_body,
        grid_spec=grid_spec,
        out_shape=jax.ShapeDtypeStruct((N_BUF, D), jnp.float32),
        compiler_params=pltpu.CompilerParams(
            dimension_semantics=("arbitrary",),
        ),
    )(te, xs, ln_g, ln_b, W1, b1, W2, b2)


@jax.jit
def kernel(x, gate_w, ln_g, ln_b, W1, b1, W2, b2):
    xf = x.reshape(BT, D)
    slot2d, wbc, base_i, aux = _router(xf, gate_w)
    tile_start = jnp.arange(N_TILES, dtype=jnp.int32)[:, None] * TILE
    te = jnp.sum((tile_start >= base_i[0][None, :]).astype(jnp.int32),
                 axis=1) - 1
    sc_scatter, sc_combine = _sc_kernels()
    xs = sc_scatter(xf, slot2d)
    ys = _ffn(te, xs, ln_g, ln_b, W1, b1, W2, b2)
    out = sc_combine(ys, slot2d, wbc, xf)
    return out.reshape(B, T, D), aux[0, 0]
